# baseline (speedup 1.0000x reference)
import jax
import jax.numpy as jnp
from jax.experimental import pallas as pl

_F32 = jnp.float32
_BS = 16
_NCHAIN = 1


def _bmm(a, b):
    return jax.lax.dot_general(a, b, (((2,), (0,)), ((), ())),
                               preferred_element_type=_F32)


def _tcg_core(ts, Wc, Wo, topk, n, d):
    bs = ts[0].shape[0]
    sims = [jax.lax.dot_general(t, t, (((2,), (2,)), ((0,), (0,))),
                                preferred_element_type=_F32) for t in ts]
    colf = jax.lax.broadcasted_iota(jnp.int32, (bs, n, n), 2).astype(_F32)
    works = list(sims)
    msels = [None] * len(ts)
    for j in range(topk):
        eqs = []
        for c in range(len(ts)):
            first = jnp.argmax(works[c], axis=2)[..., None].astype(_F32)
            eqs.append(colf == first)
        for c in range(len(ts)):
            tsel = jax.lax.dot_general(eqs[c].astype(_F32), ts[c],
                                       (((2,), (1,)), ((0,), (0,))),
                                       preferred_element_type=_F32)
            msels[c] = tsel if msels[c] is None else jnp.maximum(msels[c], tsel)
            if j < topk - 1:
                works[c] = jnp.where(eqs[c], -jnp.inf, works[c])
    pixs, touts = [], []
    for c in range(len(ts)):
        rel = msels[c] - ts[c]
        h = _bmm(ts[c], Wc[:d]) + _bmm(rel, Wc[d:])
        t_out = jax.nn.gelu(h) + ts[c]
        pixs.append(_bmm(t_out, Wo))
        touts.append(t_out)
    return pixs, touts


def _tcg_step(curs, Wp, Wc, Wo, x2s, topk, n, d):
    ts = [_bmm(c, Wp) for c in curs]
    if x2s is not None:
        ts = [t + x2 for t, x2 in zip(ts, x2s)]
    return _tcg_core(ts, Wc, Wo, topk, n, d)


def _mega_kernel(xr_ref, wexp_ref, d1p_ref, d1c_ref, d1o_ref, s1p_ref,
                 s1c_ref, s1o_ref, d2p_ref, d2c_ref, d2o_ref, s2p_ref,
                 s2c_ref, s2o_ref, w21uv_ref, s21c_ref, s21o_ref, d3p_ref,
                 d3c_ref, d3o_ref, s3p_ref, s3c_ref, s3o_ref, fc_ref,
                 bng_ref, bnb_ref, w1_ref, b1_ref, w2_ref, b2_ref, o_ref):
    bs = xr_ref.shape[0]
    hh = bs // _NCHAIN
    xr = xr_ref[...]
    curs = [jax.nn.gelu(_bmm(xr[c * hh:(c + 1) * hh], wexp_ref[...]))
            for c in range(_NCHAIN)]
    curs, x2s = _tcg_step(curs, d1p_ref[...], d1c_ref[...], d1o_ref[...],
                          None, 2, 196, 92)
    for i in range(5):
        curs, x2s = _tcg_step(curs, s1p_ref[i], s1c_ref[i], s1o_ref[i],
                              x2s, 2, 196, 92)
    curs, x2s = _tcg_step(curs, d2p_ref[...], d2c_ref[...], d2o_ref[...],
                          None, 2, 196, 192)
    for i in range(2):
        curs, x2s = _tcg_step(curs, s2p_ref[i], s2c_ref[i], s2o_ref[i],
                              x2s, 2, 196, 192)
    rowf = jax.lax.broadcasted_iota(jnp.int32, (hh, 49, 196), 1).astype(_F32)
    colq = jax.lax.broadcasted_iota(jnp.int32, (hh, 49, 196), 2).astype(_F32)
    mi = jnp.floor((rowf + 0.5) * (1.0 / 7.0))
    base = 28.0 * mi + 2.0 * (rowf - 7.0 * mi)
    gs = []
    for uv in range(4):
        off = 14.0 * (uv // 2) + (uv % 2)
        gs.append((colq == base + off).astype(_F32))
    ts = []
    for c in range(_NCHAIN):
        t = None
        for uv in range(4):
            sel = jax.lax.dot_general(gs[uv], curs[c],
                                      (((2,), (1,)), ((0,), (0,))),
                                      preferred_element_type=_F32)
            contrib = _bmm(sel, w21uv_ref[uv])
            t = contrib if t is None else t + contrib
        ts.append(t)
    curs, x2s = _tcg_core(ts, s21c_ref[...], s21o_ref[...], 2, 49, 192)
    curs, x2s = _tcg_step(curs, d3p_ref[...], d3c_ref[...], d3o_ref[...],
                          None, 9, 49, 384)
    for i in range(2):
        curs, x2s = _tcg_step(curs, s3p_ref[i], s3c_ref[i], s3o_ref[i],
                              x2s, 2, 49, 384)
    outs = []
    for c in range(_NCHAIN):
        f = _bmm(curs[c], fc_ref[...])
        f = f * bng_ref[...] + bnb_ref[...]
        f = f * jax.nn.sigmoid(f)
        fm = jnp.mean(f, axis=1)
        h2 = jax.nn.gelu(jnp.dot(fm, w1_ref[...], preferred_element_type=_F32)
                         + b1_ref[...])
        outs.append(jnp.dot(h2, w2_ref[...], preferred_element_type=_F32)
                    + b2_ref[...])
    o_ref[...] = jnp.concatenate(outs, axis=0)


def _full(shape):
    nd = len(shape)
    return pl.BlockSpec(shape, lambda b: (0,) * nd)


def _tile(shape, bs):
    nd = len(shape)
    return pl.BlockSpec((bs,) + tuple(shape[1:]),
                        lambda b: (b,) + (0,) * (nd - 1))


def kernel(x, stem_W, ds1_Wp, ds1_Wc, ds1_Wo, s1_Wp, s1_Wc, s1_Wo,
           ds2_Wp, ds2_Wc, ds2_Wo, s20_Wp, s20_Wc, s20_Wo,
           s21_Wp, s21_Wc, s21_Wo, ds3_Wp, ds3_Wc, ds3_Wo,
           s3_Wp, s3_Wc, s3_Wo, fc_W, bn_g, bn_b,
           head_W1, head_b1, head_W2, head_b2):
    B = x.shape[0]

    xr = x.reshape(B, 3, 14, 8, 14, 8).transpose(0, 2, 4, 1, 3, 5)
    xr = xr.reshape(B, 196, 192)

    w_exp = jnp.einsum('kc,ab->akcb', stem_W, jnp.eye(16, dtype=_F32))
    w_exp = w_exp.reshape(4, 4, 3, 2, 2, 46 * 16)
    w_exp = w_exp.transpose(2, 0, 3, 1, 4, 5).reshape(192, 736)

    w21uv = s21_Wp.reshape(192, 4, 192).transpose(1, 0, 2)

    args = (xr, w_exp, ds1_Wp, ds1_Wc, ds1_Wo, s1_Wp, s1_Wc, s1_Wo,
            ds2_Wp, ds2_Wc, ds2_Wo, s20_Wp, s20_Wc, s20_Wo,
            w21uv, s21_Wc, s21_Wo, ds3_Wp, ds3_Wc, ds3_Wo,
            s3_Wp, s3_Wc, s3_Wo, fc_W, bn_g.reshape(1, 384),
            bn_b.reshape(1, 384), head_W1, head_b1.reshape(1, 1536),
            head_W2, head_b2.reshape(1, 250))
    out = pl.pallas_call(
        _mega_kernel,
        grid=(B // _BS,),
        in_specs=[_tile(xr.shape, _BS)] + [_full(a.shape) for a in args[1:]],
        out_specs=_tile((B, 250), _BS),
        out_shape=jax.ShapeDtypeStruct((B, 250), _F32),
    )(*args)
    return out


# speedup vs baseline: 1.1469x; 1.1469x over previous
import jax
import jax.numpy as jnp
from jax.experimental import pallas as pl

_F32 = jnp.float32
_BS = 16


def _bmm(a, b):
    return jax.lax.dot_general(a, b, (((2,), (0,)), ((), ())),
                               preferred_element_type=_F32)


def _tcg_core(t, Wca, Wcb, topk, n):
    bs = t.shape[0]
    sim = jax.lax.dot_general(t, t, (((2,), (2,)), ((0,), (0,))),
                              preferred_element_type=_F32)
    colf = jax.lax.broadcasted_iota(jnp.int32, (bs, n, n), 2).astype(_F32)
    work = sim
    msel = None
    for j in range(topk):
        rowmax = jnp.max(work, axis=2, keepdims=True)
        first = jnp.min(jnp.where(work == rowmax, colf, float(n)),
                        axis=2, keepdims=True)
        eq = colf == first
        tsel = jax.lax.dot_general(eq.astype(_F32), t,
                                   (((2,), (1,)), ((0,), (0,))),
                                   preferred_element_type=_F32)
        msel = tsel if msel is None else jnp.maximum(msel, tsel)
        if j < topk - 1:
            work = jnp.where(eq, -jnp.inf, work)
    h = _bmm(t, Wca) + _bmm(msel, Wcb)
    return jax.nn.gelu(h) + t


def _mega_kernel(xr_ref, wexp_ref, a1_ref, wca_ref, wcb_ref, m196_ref,
                 m192a_ref, m192b_ref, wca192_ref, wcb192_ref, w21uv_ref,
                 wca21a_ref, wca21b_ref, m49a_ref, m49b_ref, wca3_ref,
                 wcb3_ref, d3ca_ref, d3cb_ref, fc_ref,
                 bnb_ref, w1_ref, b1_ref, w2_ref, b2_ref, o_ref):
    bs = xr_ref.shape[0]
    cur0 = jax.nn.gelu(_bmm(xr_ref[...], wexp_ref[...]))
    t_out = _tcg_core(_bmm(cur0, a1_ref[...]), wca_ref[0], wcb_ref[0],
                      2, 196)
    for i in range(5):
        t_out = _tcg_core(_bmm(t_out, m196_ref[i]), wca_ref[i + 1],
                          wcb_ref[i + 1], 2, 196)
    t_out = _tcg_core(_bmm(t_out, m192a_ref[...]), wca192_ref[0],
                      wcb192_ref[0], 2, 196)
    for i in range(2):
        t_out = _tcg_core(_bmm(t_out, m192b_ref[i]), wca192_ref[i + 1],
                          wcb192_ref[i + 1], 2, 196)
    rowf = jax.lax.broadcasted_iota(jnp.int32, (bs, 49, 196), 1).astype(_F32)
    colq = jax.lax.broadcasted_iota(jnp.int32, (bs, 49, 196), 2).astype(_F32)
    mi = jnp.floor((rowf + 0.5) * (1.0 / 7.0))
    base = 28.0 * mi + 2.0 * (rowf - 7.0 * mi)
    t = None
    for uv in range(4):
        off = 14.0 * (uv // 2) + (uv % 2)
        g = (colq == base + off).astype(_F32)
        sel = jax.lax.dot_general(g, t_out, (((2,), (1,)), ((0,), (0,))),
                                  preferred_element_type=_F32)
        contrib = _bmm(sel, w21uv_ref[uv])
        t = contrib if t is None else t + contrib
    t_out = _tcg_core(t, wca21a_ref[...], wca21b_ref[...], 2, 49)
    t_out = _tcg_core(_bmm(t_out, m49a_ref[...]), d3ca_ref[...],
                      d3cb_ref[...], 9, 49)
    for i in range(2):
        t_out = _tcg_core(_bmm(t_out, m49b_ref[i]), wca3_ref[i],
                          wcb3_ref[i], 2, 49)
    f = _bmm(t_out, fc_ref[...]) + bnb_ref[...]
    f = f * jax.nn.sigmoid(f)
    fm = jnp.mean(f, axis=1)
    h2 = jax.nn.gelu(jnp.dot(fm, w1_ref[...], preferred_element_type=_F32)
                     + b1_ref[...])
    o_ref[...] = jnp.dot(h2, w2_ref[...], preferred_element_type=_F32) + b2_ref[...]


def _full(shape):
    nd = len(shape)
    return pl.BlockSpec(shape, lambda b: (0,) * nd)


def _tile(shape, bs):
    nd = len(shape)
    return pl.BlockSpec((bs,) + tuple(shape[1:]),
                        lambda b: (b,) + (0,) * (nd - 1))


def kernel(x, stem_W, ds1_Wp, ds1_Wc, ds1_Wo, s1_Wp, s1_Wc, s1_Wo,
           ds2_Wp, ds2_Wc, ds2_Wo, s20_Wp, s20_Wc, s20_Wo,
           s21_Wp, s21_Wc, s21_Wo, ds3_Wp, ds3_Wc, ds3_Wo,
           s3_Wp, s3_Wc, s3_Wo, fc_W, bn_g, bn_b,
           head_W1, head_b1, head_W2, head_b2):
    B = x.shape[0]

    xr = x.reshape(B, 3, 14, 8, 14, 8).transpose(0, 2, 4, 1, 3, 5)
    xr = xr.reshape(B, 196, 192)

    w_exp = jnp.einsum('kc,ab->akcb', stem_W, jnp.eye(16, dtype=_F32))
    w_exp = w_exp.reshape(4, 4, 3, 2, 2, 46 * 16)
    w_exp = w_exp.transpose(2, 0, 3, 1, 4, 5).reshape(192, 736)

    eye92 = jnp.eye(92, dtype=_F32)
    eye192 = jnp.eye(192, dtype=_F32)
    eye384 = jnp.eye(384, dtype=_F32)

    wo_prev = [ds1_Wo] + [s1_Wo[i] for i in range(4)]
    m196 = jnp.stack([wo_prev[i] @ s1_Wp[i] + eye92 for i in range(5)])
    wcs = [ds1_Wc] + [s1_Wc[i] for i in range(5)]
    wca = jnp.stack([w[:92] - w[92:] for w in wcs])
    wcb = jnp.stack([w[92:] for w in wcs])
    m192a = s1_Wo[4] @ ds2_Wp
    m192b = jnp.stack([ds2_Wo @ s20_Wp[0] + eye192,
                       s20_Wo[0] @ s20_Wp[1] + eye192])
    wcs192 = [ds2_Wc, s20_Wc[0], s20_Wc[1]]
    wca192 = jnp.stack([w[:192] - w[192:] for w in wcs192])
    wcb192 = jnp.stack([w[192:] for w in wcs192])
    w21uv = s21_Wp.reshape(192, 4, 192).transpose(1, 0, 2)
    w21uv = jnp.einsum('ck,ukd->ucd', s20_Wo[1], w21uv)
    wca21a = s21_Wc[:192] - s21_Wc[192:]
    wca21b = s21_Wc[192:]
    m49a = s21_Wo @ ds3_Wp
    m49b = jnp.stack([ds3_Wo @ s3_Wp[0] + eye384,
                      s3_Wo[0] @ s3_Wp[1] + eye384])
    d3ca = ds3_Wc[:384] - ds3_Wc[384:]
    d3cb = ds3_Wc[384:]
    wca3 = jnp.stack([s3_Wc[i][:384] - s3_Wc[i][384:] for i in range(2)])
    wcb3 = jnp.stack([s3_Wc[i][384:] for i in range(2)])
    fc = (s3_Wo[1] @ fc_W) * bn_g[None, :]

    args = (xr, w_exp, ds1_Wp, wca, wcb, m196, m192a, m192b, wca192, wcb192,
            w21uv, wca21a, wca21b, m49a, m49b, wca3, wcb3, d3ca, d3cb, fc,
            bn_b.reshape(1, 384), head_W1, head_b1.reshape(1, 1536),
            head_W2, head_b2.reshape(1, 250))
    out = pl.pallas_call(
        _mega_kernel,
        grid=(B // _BS,),
        in_specs=[_tile(xr.shape, _BS)] + [_full(a.shape) for a in args[1:]],
        out_specs=_tile((B, 250), _BS),
        out_shape=jax.ShapeDtypeStruct((B, 250), _F32),
    )(*args)
    return out
